# baseline (speedup 1.0000x reference)
import functools

import jax
import jax.numpy as jnp
from jax import lax
from jax.experimental import pallas as pl
from jax.experimental.pallas import tpu as pltpu
from jax.experimental.pallas import tpu_sc as plsc

_EPS2 = 1e-6
_NBD = 2048
_NBC = 512
_IMAX = 2**31 - 1


def _argmin_body(p8_ref, mt_ref, iota_ref, bestd_ref, besti_ref):
    i = pl.program_id(0)
    nb = mt_ref.shape[1]
    mt = mt_ref[...]
    s = jnp.dot(p8_ref[...], mt, preferred_element_type=jnp.float32)
    msqc = jnp.sum(mt * mt, axis=0, keepdims=True)
    d2c = s + msqc
    dmin = jnp.min(d2c, axis=1, keepdims=True)
    key = jax.lax.bitcast_convert_type(d2c, jnp.int32)
    key = (key & ~(nb - 1)) | iota_ref[...]
    cand = (jnp.min(key, axis=1, keepdims=True) & (nb - 1)) + i * nb

    @pl.when(i == 0)
    def _():
        bestd_ref[...] = dmin
        besti_ref[...] = cand

    @pl.when(i > 0)
    def _():
        prev = bestd_ref[...]
        better = dmin < prev
        bestd_ref[...] = jnp.where(better, dmin, prev)
        besti_ref[...] = jnp.where(better, cand, besti_ref[...])


def _address_body(bestd_ref, besti_ref, u_ref, p8_ref, cin_ref,
                  lw_ref, free_ref):
    b = bestd_ref.shape[0]
    gc, _, nbc = u_ref.shape
    f32 = jnp.float32

    umin = jnp.min(u_ref[...])
    free_ref[...] = jnp.zeros_like(free_ref)
    tri = (jax.lax.broadcasted_iota(jnp.int32, (nbc, nbc), 0)
           <= jax.lax.broadcasted_iota(jnp.int32, (nbc, nbc), 1)
           ).astype(f32)
    rio = jax.lax.broadcasted_iota(jnp.int32, (b, nbc), 0).astype(f32)
    gj8 = jax.lax.broadcasted_iota(jnp.int32, (nbc, 8), 0).astype(f32)

    def body(j, c0):
        m = u_ref[j] == umin

        @pl.when(c0 < b)
        def _():
            mf = m.astype(f32)
            pos = jnp.dot(mf, tri, preferred_element_type=f32)
            pos = pos + c0.astype(f32)
            a = jnp.where((rio + 1.0 == jnp.broadcast_to(pos, (b, nbc)))
                          & jnp.broadcast_to(m, (b, nbc)), 1.0, 0.0)
            gj = gj8 + (j * nbc).astype(f32)
            free_ref[...] = free_ref[...] + jnp.dot(
                a, gj, preferred_element_type=f32,
                precision=jax.lax.Precision.HIGHEST)

        return c0 + jnp.sum(m.astype(jnp.int32))

    jax.lax.fori_loop(0, gc, body, jnp.int32(0))

    eye = (jax.lax.broadcasted_iota(jnp.int32, (b, b), 0)
           == jax.lax.broadcasted_iota(jnp.int32, (b, b), 1)).astype(f32)
    iot0 = jax.lax.broadcasted_iota(jnp.int32, (b, b), 0).astype(f32)
    iot1 = jax.lax.broadcasted_iota(jnp.int32, (b, b), 1).astype(f32)

    p8 = p8_ref[...]
    psq = jnp.sum(p8 * p8, axis=1, keepdims=True) * 0.25
    bestdc = bestd_ref[...]
    cbc = jnp.broadcast_to(cin_ref[0:1, 0:1], (b, 1))
    mask = bestdc > cbc - psq + 2e-5
    mf = mask.astype(f32)
    ltri = (iot1 <= iot0)
    cum = jnp.dot(ltri.astype(f32), mf, preferred_element_type=f32)
    rank = jnp.clip(cum - 1.0, 0.0, float(b - 1))
    o1 = (iot1 == jnp.broadcast_to(rank, (b, b))).astype(f32)
    fsel8 = jnp.dot(o1, free_ref[...], preferred_element_type=f32,
                    precision=jax.lax.Precision.HIGHEST)
    idx = jnp.where(mask, fsel8[:, 0:1], besti_ref[...].astype(f32))
    idx_row = jnp.sum(eye * jnp.broadcast_to(idx, (b, b)), axis=0,
                      keepdims=True)
    e = jnp.broadcast_to(idx, (b, b)) == jnp.broadcast_to(idx_row, (b, b))
    lw_row = jnp.max(jnp.where(e, iot0, -1.0), axis=0, keepdims=True)
    lw_col = jnp.sum(eye * jnp.broadcast_to(lw_row, (b, b)), axis=1,
                     keepdims=True)
    lw_ref[...] = lw_col.astype(jnp.int32)


@jax.jit
def kernel(points, descriptors, mem_points, mem_descriptors, usage):
    del mem_descriptors
    b = points.shape[0]
    n = mem_points.shape[0]
    f = descriptors.shape[1]
    gd = (n + _NBD - 1) // _NBD
    npad = gd * _NBD
    gc = npad // _NBC

    psq = jnp.sum(points * points, axis=1)
    khalf = jnp.ceil(jnp.log2(jnp.max(psq) + 2.0) * 0.5)
    cshift = jnp.exp2(2.0 * khalf)
    mt = jnp.full((8, npad), 0.0, jnp.float32)
    mt = mt.at[:3, :n].set(mem_points.T).at[:3, n:].set(1e18)
    mt = mt.at[3, :].set(jnp.exp2(khalf))
    p8 = jnp.zeros((b, 8), jnp.float32).at[:, :3].set(points * -2.0)
    u_r = jnp.full((npad,), _IMAX, jnp.int32).at[:n].set(usage).reshape(
        gc, 1, _NBC)
    iota_c = jnp.broadcast_to(jnp.arange(_NBD, dtype=jnp.int32)[None, :],
                              (b, _NBD))
    cin = jnp.full((1, 128), cshift, jnp.float32)

    bestd, besti = pl.pallas_call(
        _argmin_body,
        grid=(gd,),
        in_specs=[
            pl.BlockSpec((b, 8), lambda i: (0, 0)),
            pl.BlockSpec((8, _NBD), lambda i: (0, i)),
            pl.BlockSpec((b, _NBD), lambda i: (0, 0)),
        ],
        out_specs=[
            pl.BlockSpec((b, 1), lambda i: (0, 0)),
            pl.BlockSpec((b, 1), lambda i: (0, 0)),
        ],
        out_shape=[
            jax.ShapeDtypeStruct((b, 1), jnp.float32),
            jax.ShapeDtypeStruct((b, 1), jnp.int32),
        ],
    )(p8, mt, iota_c)

    lw = pl.pallas_call(
        _address_body,
        in_specs=[pl.BlockSpec(x.shape, lambda nd=x.ndim: (0,) * nd)
                  for x in (bestd, besti, u_r, p8, cin)],
        out_specs=pl.BlockSpec((b, 1), lambda: (0, 0)),
        out_shape=jax.ShapeDtypeStruct((b, 1), jnp.int32),
        scratch_shapes=[pltpu.VMEM((b, 8), jnp.float32)],
    )(bestd, besti, u_r, p8, cin)

    return _sc_gather(b, f)(lw.reshape(b), descriptors)


@functools.lru_cache(maxsize=None)
def _sc_gather(b, f):
    info = plsc.get_sparse_core_info()
    nw = info.num_cores * info.num_subcores
    bw = b // nw
    mesh = plsc.VectorSubcoreMesh(core_axis_name="c", subcore_axis_name="s")

    @functools.partial(
        pl.kernel, mesh=mesh,
        out_type=jax.ShapeDtypeStruct((b, f), jnp.float32),
        scratch_types=[
            pltpu.VMEM((bw,), jnp.int32),
            pltpu.VMEM((bw, f), jnp.float32),
            pltpu.SemaphoreType.DMA,
        ],
    )
    def k(idx_hbm, table_hbm, out_hbm, idx_v, rows_v, sem):
        wid = lax.axis_index("s") * info.num_cores + lax.axis_index("c")
        base = wid * bw
        pltpu.sync_copy(idx_hbm.at[pl.ds(base, bw)], idx_v)
        pltpu.async_copy(table_hbm.at[idx_v], rows_v, sem).wait()
        pltpu.sync_copy(rows_v, out_hbm.at[pl.ds(base, bw)])

    return k


# speedup vs baseline: 1.0563x; 1.0563x over previous
import functools

import jax
import jax.numpy as jnp
from jax import lax
from jax.experimental import pallas as pl
from jax.experimental.pallas import tpu as pltpu
from jax.experimental.pallas import tpu_sc as plsc

_EPS2 = 1e-6
_NBD = 4096
_NBC = 512
_IMAX = 2**31 - 1


def _fused_body(p8_ref, mt_ref, iota_ref, u_ref, cin_ref, lw_ref,
                bestd_ref, besti_ref, free_ref):
    i = pl.program_id(0)
    gd = pl.num_programs(0) - 1
    nb = mt_ref.shape[1]
    b = p8_ref.shape[0]
    f32 = jnp.float32

    @pl.when(i < gd)
    def _():
        mt = mt_ref[...]
        s = jnp.dot(p8_ref[...], mt, preferred_element_type=f32)
        msqc = jnp.sum(mt * mt, axis=0, keepdims=True)
        d2c = s + msqc
        dmin = jnp.min(d2c, axis=1, keepdims=True)
        key = jax.lax.bitcast_convert_type(d2c, jnp.int32)
        key = (key & ~(nb - 1)) | iota_ref[...]
        cand = (jnp.min(key, axis=1, keepdims=True) & (nb - 1)) + i * nb

        @pl.when(i == 0)
        def _():
            bestd_ref[...] = dmin
            besti_ref[...] = cand

        @pl.when(i > 0)
        def _():
            prev = bestd_ref[...]
            better = dmin < prev
            bestd_ref[...] = jnp.where(better, dmin, prev)
            besti_ref[...] = jnp.where(better, cand, besti_ref[...])

    @pl.when(i == gd)
    def _():
        gc, _, nbc = u_ref.shape

        umin = jnp.min(u_ref[...])
        free_ref[...] = jnp.zeros_like(free_ref)
        tri = (jax.lax.broadcasted_iota(jnp.int32, (nbc, nbc), 0)
               <= jax.lax.broadcasted_iota(jnp.int32, (nbc, nbc), 1)
               ).astype(f32)
        rio = jax.lax.broadcasted_iota(jnp.int32, (b, nbc), 0).astype(f32)
        gj8 = jax.lax.broadcasted_iota(jnp.int32, (nbc, 8), 0).astype(f32)

        def body(j, c0):
            m = u_ref[j] == umin

            @pl.when(c0 < b)
            def _():
                mf = m.astype(f32)
                pos = jnp.dot(mf, tri, preferred_element_type=f32)
                pos = pos + c0.astype(f32)
                a = jnp.where((rio + 1.0 == jnp.broadcast_to(pos, (b, nbc)))
                              & jnp.broadcast_to(m, (b, nbc)), 1.0, 0.0)
                gj = gj8 + (j * nbc).astype(f32)
                free_ref[...] = free_ref[...] + jnp.dot(
                    a, gj, preferred_element_type=f32,
                    precision=jax.lax.Precision.HIGHEST)

            return c0 + jnp.sum(m.astype(jnp.int32))

        jax.lax.fori_loop(0, gc, body, jnp.int32(0))

        eye = (jax.lax.broadcasted_iota(jnp.int32, (b, b), 0)
               == jax.lax.broadcasted_iota(jnp.int32, (b, b), 1)).astype(f32)
        iot0 = jax.lax.broadcasted_iota(jnp.int32, (b, b), 0).astype(f32)
        iot1 = jax.lax.broadcasted_iota(jnp.int32, (b, b), 1).astype(f32)

        p8 = p8_ref[...]
        psq = jnp.sum(p8 * p8, axis=1, keepdims=True) * 0.25
        cbc = jnp.broadcast_to(cin_ref[0:1, 0:1], (b, 1))
        mask = bestd_ref[...] > cbc - psq + 2e-5
        mf = mask.astype(f32)
        ltri = (iot1 <= iot0)
        cum = jnp.dot(ltri.astype(f32), mf, preferred_element_type=f32)
        rank = jnp.clip(cum - 1.0, 0.0, float(b - 1))
        o1 = (iot1 == jnp.broadcast_to(rank, (b, b))).astype(f32)
        fsel8 = jnp.dot(o1, free_ref[...], preferred_element_type=f32,
                        precision=jax.lax.Precision.HIGHEST)
        idx = jnp.where(mask, fsel8[:, 0:1], besti_ref[...].astype(f32))
        idx_row = jnp.sum(eye * jnp.broadcast_to(idx, (b, b)), axis=0,
                          keepdims=True)
        e = (jnp.broadcast_to(idx, (b, b))
             == jnp.broadcast_to(idx_row, (b, b)))
        lw_row = jnp.max(jnp.where(e, iot0, -1.0), axis=0, keepdims=True)
        lw_col = jnp.sum(eye * jnp.broadcast_to(lw_row, (b, b)), axis=1,
                         keepdims=True)
        lw_ref[...] = lw_col.astype(jnp.int32)


@functools.lru_cache(maxsize=None)
def _sc_gather(b, f):
    info = plsc.get_sparse_core_info()
    nw = info.num_cores * info.num_subcores
    bw = b // nw
    mesh = plsc.VectorSubcoreMesh(core_axis_name="c", subcore_axis_name="s")

    @functools.partial(
        pl.kernel, mesh=mesh,
        out_type=jax.ShapeDtypeStruct((b, f), jnp.float32),
        scratch_types=[
            pltpu.VMEM((bw,), jnp.int32),
            pltpu.VMEM((bw, f), jnp.float32),
            pltpu.SemaphoreType.DMA,
        ],
    )
    def k(idx_hbm, table_hbm, out_hbm, idx_v, rows_v, sem):
        wid = lax.axis_index("s") * info.num_cores + lax.axis_index("c")
        base = wid * bw
        pltpu.sync_copy(idx_hbm.at[pl.ds(base, bw)], idx_v)
        pltpu.async_copy(table_hbm.at[idx_v], rows_v, sem).wait()
        pltpu.sync_copy(rows_v, out_hbm.at[pl.ds(base, bw)])

    return k


@jax.jit
def kernel(points, descriptors, mem_points, mem_descriptors, usage):
    del mem_descriptors
    b = points.shape[0]
    n = mem_points.shape[0]
    f = descriptors.shape[1]
    gd = (n + _NBD - 1) // _NBD
    npad = gd * _NBD
    gc = npad // _NBC

    psq = jnp.sum(points * points, axis=1)
    khalf = jnp.ceil(jnp.log2(jnp.max(psq) + 2.0) * 0.5)
    cshift = jnp.exp2(2.0 * khalf)
    mt = jnp.full((8, npad), 0.0, jnp.float32)
    mt = mt.at[:3, :n].set(mem_points.T).at[:3, n:].set(1e18)
    mt = mt.at[3, :].set(jnp.exp2(khalf))
    p8 = jnp.zeros((b, 8), jnp.float32).at[:, :3].set(points * -2.0)
    u_r = jnp.full((npad,), _IMAX, jnp.int32).at[:n].set(usage).reshape(
        gc, 1, _NBC)
    iota_c = jnp.broadcast_to(jnp.arange(_NBD, dtype=jnp.int32)[None, :],
                              (b, _NBD))
    cin = jnp.full((1, 128), cshift, jnp.float32)

    lw = pl.pallas_call(
        _fused_body,
        grid=(gd + 1,),
        in_specs=[
            pl.BlockSpec((b, 8), lambda i: (0, 0)),
            pl.BlockSpec((8, _NBD), lambda i: (0, jnp.minimum(i, gd - 1))),
            pl.BlockSpec((b, _NBD), lambda i: (0, 0)),
            pl.BlockSpec((gc, 1, _NBC), lambda i: (0, 0, 0)),
            pl.BlockSpec((1, 128), lambda i: (0, 0)),
        ],
        out_specs=pl.BlockSpec((b, 1), lambda i: (0, 0)),
        out_shape=jax.ShapeDtypeStruct((b, 1), jnp.int32),
        scratch_shapes=[
            pltpu.VMEM((b, 1), jnp.float32),
            pltpu.VMEM((b, 1), jnp.int32),
            pltpu.VMEM((b, 8), jnp.float32),
        ],
    )(p8, mt, iota_c, u_r, cin)

    return _sc_gather(b, f)(lw.reshape(b), descriptors)


# speedup vs baseline: 1.1211x; 1.0613x over previous
import functools

import jax
import jax.numpy as jnp
from jax import lax
from jax.experimental import pallas as pl
from jax.experimental.pallas import tpu as pltpu
from jax.experimental.pallas import tpu_sc as plsc

_EPS2 = 1e-6
_NBD = 4096
_NBC = 512
_IMAX = 2**31 - 1


def _fused_body(p8_ref, mt_ref, iota_ref, u_ref, cin_ref, lw_ref,
                bestd_ref, besti_ref, free_ref):
    i = pl.program_id(0)
    gd = pl.num_programs(0) - 1
    nb = mt_ref.shape[1]
    b = p8_ref.shape[0]
    f32 = jnp.float32

    @pl.when(i < gd)
    def _():
        mt = mt_ref[...]
        s = jnp.dot(p8_ref[...], mt, preferred_element_type=f32)
        msqc = jnp.sum(mt * mt, axis=0, keepdims=True)
        d2c = s + msqc
        dmin = jnp.min(d2c, axis=1, keepdims=True)
        key = jax.lax.bitcast_convert_type(d2c, jnp.int32)
        key = (key & ~(nb - 1)) | iota_ref[...]
        cand = (jnp.min(key, axis=1, keepdims=True) & (nb - 1)) + i * nb

        @pl.when(i == 0)
        def _():
            bestd_ref[...] = dmin
            besti_ref[...] = cand

        @pl.when(i > 0)
        def _():
            prev = bestd_ref[...]
            better = dmin < prev
            bestd_ref[...] = jnp.where(better, dmin, prev)
            besti_ref[...] = jnp.where(better, cand, besti_ref[...])

    @pl.when(i == gd)
    def _():
        gc, _, nbc = u_ref.shape

        umin = jnp.min(u_ref[...])
        free_ref[...] = jnp.zeros_like(free_ref)
        tri = (jax.lax.broadcasted_iota(jnp.int32, (nbc, nbc), 0)
               <= jax.lax.broadcasted_iota(jnp.int32, (nbc, nbc), 1)
               ).astype(f32)
        rio = jax.lax.broadcasted_iota(jnp.int32, (b, nbc), 0).astype(f32)
        gj8 = jax.lax.broadcasted_iota(jnp.int32, (nbc, 8), 0).astype(f32)

        def body(j, c0):
            m = u_ref[j] == umin

            @pl.when(c0 < b)
            def _():
                mf = m.astype(f32)
                pos = jnp.dot(mf, tri, preferred_element_type=f32)
                pos = pos + c0.astype(f32)
                a = jnp.where((rio + 1.0 == jnp.broadcast_to(pos, (b, nbc)))
                              & jnp.broadcast_to(m, (b, nbc)), 1.0, 0.0)
                gj = gj8 + (j * nbc).astype(f32)
                free_ref[...] = free_ref[...] + jnp.dot(
                    a, gj, preferred_element_type=f32,
                    precision=jax.lax.Precision.HIGHEST)

            return c0 + jnp.sum(m.astype(jnp.int32))

        jax.lax.fori_loop(0, gc, body, jnp.int32(0))

        eye = (jax.lax.broadcasted_iota(jnp.int32, (b, b), 0)
               == jax.lax.broadcasted_iota(jnp.int32, (b, b), 1)).astype(f32)
        iot0 = jax.lax.broadcasted_iota(jnp.int32, (b, b), 0).astype(f32)
        iot1 = jax.lax.broadcasted_iota(jnp.int32, (b, b), 1).astype(f32)

        p8 = p8_ref[...]
        psq = jnp.sum(p8 * p8, axis=1, keepdims=True) * 0.25
        cbc = jnp.broadcast_to(cin_ref[0:1, 0:1], (b, 1))
        mask = bestd_ref[...] > cbc - psq + 2e-5
        mf = mask.astype(f32)
        ltri = (iot1 <= iot0)
        cum = jnp.dot(ltri.astype(f32), mf, preferred_element_type=f32)
        rank = jnp.clip(cum - 1.0, 0.0, float(b - 1))
        o1 = (iot1 == jnp.broadcast_to(rank, (b, b))).astype(f32)
        fsel8 = jnp.dot(o1, free_ref[...], preferred_element_type=f32,
                        precision=jax.lax.Precision.HIGHEST)
        idx = jnp.where(mask, fsel8[:, 0:1], besti_ref[...].astype(f32))
        idx_row = jnp.sum(eye * jnp.broadcast_to(idx, (b, b)), axis=0,
                          keepdims=True)
        e = (jnp.broadcast_to(idx, (b, b))
             == jnp.broadcast_to(idx_row, (b, b)))
        lw_row = jnp.max(jnp.where(e, iot0, -1.0), axis=0, keepdims=True)
        lw_col = jnp.sum(eye * jnp.broadcast_to(lw_row, (b, b)), axis=1,
                         keepdims=True)
        lw_ref[...] = lw_col.astype(jnp.int32)


@functools.lru_cache(maxsize=None)
def _sc_gather(b, f):
    info = plsc.get_sparse_core_info()
    nw = info.num_cores * info.num_subcores
    bw = b // nw
    mesh = plsc.VectorSubcoreMesh(core_axis_name="c", subcore_axis_name="s")

    @functools.partial(
        pl.kernel, mesh=mesh,
        out_type=jax.ShapeDtypeStruct((b, f), jnp.float32),
        scratch_types=[
            pltpu.VMEM((bw,), jnp.int32),
            pltpu.VMEM((bw, f), jnp.float32),
            pltpu.SemaphoreType.DMA,
        ],
    )
    def k(idx_hbm, table_hbm, out_hbm, idx_v, rows_v, sem):
        wid = lax.axis_index("s") * info.num_cores + lax.axis_index("c")
        base = wid * bw
        pltpu.sync_copy(idx_hbm.at[pl.ds(base, bw)], idx_v)
        pltpu.async_copy(table_hbm.at[idx_v], rows_v, sem).wait()
        pltpu.sync_copy(rows_v, out_hbm.at[pl.ds(base, bw)])

    return k


@jax.jit
def kernel(points, descriptors, mem_points, mem_descriptors, usage):
    del mem_descriptors
    b = points.shape[0]
    n = mem_points.shape[0]
    f = descriptors.shape[1]
    gd = (n + _NBD - 1) // _NBD
    npad = gd * _NBD
    gc = npad // _NBC

    psq = jnp.sum(points * points, axis=1)
    khalf = jnp.ceil(jnp.log2(jnp.max(psq) + 2.0) * 0.5)
    cshift = jnp.exp2(2.0 * khalf)
    mt = jnp.full((8, npad), 0.0, jnp.float32)
    mt = mt.at[:3, :n].set(mem_points.T).at[:3, n:].set(1e18)
    mt = mt.at[3, :].set(jnp.exp2(khalf))
    p8 = jnp.zeros((b, 8), jnp.float32).at[:, :3].set(points * -2.0)
    u_r = jnp.full((npad,), _IMAX, jnp.int32).at[:n].set(usage).reshape(
        gc, 1, _NBC)
    iota_c = jnp.arange(_NBD, dtype=jnp.int32)[None, :]
    cin = jnp.full((1, 128), cshift, jnp.float32)

    lw = pl.pallas_call(
        _fused_body,
        grid=(gd + 1,),
        in_specs=[
            pl.BlockSpec((b, 8), lambda i: (0, 0)),
            pl.BlockSpec((8, _NBD), lambda i: (0, jnp.minimum(i, gd - 1))),
            pl.BlockSpec((1, _NBD), lambda i: (0, 0)),
            pl.BlockSpec((gc, 1, _NBC), lambda i: (0, 0, 0)),
            pl.BlockSpec((1, 128), lambda i: (0, 0)),
        ],
        out_specs=pl.BlockSpec((b, 1), lambda i: (0, 0)),
        out_shape=jax.ShapeDtypeStruct((b, 1), jnp.int32),
        scratch_shapes=[
            pltpu.VMEM((b, 1), jnp.float32),
            pltpu.VMEM((b, 1), jnp.int32),
            pltpu.VMEM((b, 8), jnp.float32),
        ],
    )(p8, mt, iota_c, u_r, cin)

    return _sc_gather(b, f)(lw.reshape(b), descriptors)


# speedup vs baseline: 1.1216x; 1.0004x over previous
import functools

import jax
import jax.numpy as jnp
from jax import lax
from jax.experimental import pallas as pl
from jax.experimental.pallas import tpu as pltpu
from jax.experimental.pallas import tpu_sc as plsc

_EPS2 = 1e-6
_NBD = 4096
_NBC = 512
_IMAX = 2**31 - 1


def _fused_body(p8_ref, mt_ref, iota_ref, u_ref, cin_ref, lw_ref,
                bestd_ref, besti_ref, free_ref):
    i = pl.program_id(0)
    gd = pl.num_programs(0) - 1
    nb = mt_ref.shape[1]
    b = p8_ref.shape[0]
    f32 = jnp.float32

    @pl.when(i < gd)
    def _():
        mt = mt_ref[...]
        s = jnp.dot(p8_ref[...], mt, preferred_element_type=f32)
        msqc = jnp.sum(mt * mt, axis=0, keepdims=True)
        d2c = s + msqc
        dmin = jnp.min(d2c, axis=1, keepdims=True)
        key = jax.lax.bitcast_convert_type(d2c, jnp.int32)
        key = (key & ~(nb - 1)) | iota_ref[...]
        cand = (jnp.min(key, axis=1, keepdims=True) & (nb - 1)) + i * nb

        @pl.when(i == 0)
        def _():
            bestd_ref[...] = dmin
            besti_ref[...] = cand

        @pl.when(i > 0)
        def _():
            prev = bestd_ref[...]
            better = dmin < prev
            bestd_ref[...] = jnp.where(better, dmin, prev)
            besti_ref[...] = jnp.where(better, cand, besti_ref[...])

    @pl.when(i == gd)
    def _():
        gc, _, nbc = u_ref.shape

        umin = jnp.min(u_ref[...])
        free_ref[...] = jnp.zeros_like(free_ref)
        tri = (jax.lax.broadcasted_iota(jnp.int32, (nbc, nbc), 0)
               <= jax.lax.broadcasted_iota(jnp.int32, (nbc, nbc), 1)
               ).astype(f32)
        rio = jax.lax.broadcasted_iota(jnp.int32, (b, nbc), 0).astype(f32)
        gj8 = jax.lax.broadcasted_iota(jnp.int32, (nbc, 8), 0).astype(f32)

        def body(j, c0):
            m = u_ref[j] == umin

            @pl.when(c0 < b)
            def _():
                mf = m.astype(f32)
                pos = jnp.dot(mf, tri, preferred_element_type=f32)
                pos = pos + c0.astype(f32)
                a = jnp.where((rio + 1.0 == jnp.broadcast_to(pos, (b, nbc)))
                              & jnp.broadcast_to(m, (b, nbc)), 1.0, 0.0)
                gj = gj8 + (j * nbc).astype(f32)
                free_ref[...] = free_ref[...] + jnp.dot(
                    a, gj, preferred_element_type=f32,
                    precision=jax.lax.Precision.HIGHEST)

            return c0 + jnp.sum(m.astype(jnp.int32))

        jax.lax.fori_loop(0, gc, body, jnp.int32(0))

        eye = (jax.lax.broadcasted_iota(jnp.int32, (b, b), 0)
               == jax.lax.broadcasted_iota(jnp.int32, (b, b), 1)).astype(f32)
        iot0 = jax.lax.broadcasted_iota(jnp.int32, (b, b), 0).astype(f32)
        iot1 = jax.lax.broadcasted_iota(jnp.int32, (b, b), 1).astype(f32)

        p8 = p8_ref[...]
        psq = jnp.sum(p8 * p8, axis=1, keepdims=True) * 0.25
        cbc = jnp.broadcast_to(cin_ref[0:1, 0:1], (b, 1))
        mask = bestd_ref[...] > cbc - psq + 2e-5
        mf = mask.astype(f32)
        ltri = (iot1 <= iot0)
        cum = jnp.dot(ltri.astype(f32), mf, preferred_element_type=f32)
        rank = jnp.clip(cum - 1.0, 0.0, float(b - 1))
        o1 = (iot1 == jnp.broadcast_to(rank, (b, b))).astype(f32)
        fsel8 = jnp.dot(o1, free_ref[...], preferred_element_type=f32,
                        precision=jax.lax.Precision.HIGHEST)
        idx = jnp.where(mask, fsel8[:, 0:1], besti_ref[...].astype(f32))
        idx_row = jnp.sum(eye * jnp.broadcast_to(idx, (b, b)), axis=0,
                          keepdims=True)
        e = (jnp.broadcast_to(idx, (b, b))
             == jnp.broadcast_to(idx_row, (b, b)))
        lw_col = jnp.max(jnp.where(e, iot1, -1.0), axis=1, keepdims=True)
        lw_ref[...] = lw_col.astype(jnp.int32)


@functools.lru_cache(maxsize=None)
def _sc_gather(b, f):
    info = plsc.get_sparse_core_info()
    nw = info.num_cores * info.num_subcores
    bw = b // nw
    mesh = plsc.VectorSubcoreMesh(core_axis_name="c", subcore_axis_name="s")

    @functools.partial(
        pl.kernel, mesh=mesh,
        out_type=jax.ShapeDtypeStruct((b, f), jnp.float32),
        scratch_types=[
            pltpu.VMEM((bw,), jnp.int32),
            pltpu.VMEM((bw, f), jnp.float32),
            pltpu.SemaphoreType.DMA,
        ],
    )
    def k(idx_hbm, table_hbm, out_hbm, idx_v, rows_v, sem):
        wid = lax.axis_index("s") * info.num_cores + lax.axis_index("c")
        base = wid * bw
        pltpu.sync_copy(idx_hbm.at[pl.ds(base, bw)], idx_v)
        pltpu.async_copy(table_hbm.at[idx_v], rows_v, sem).wait()
        pltpu.sync_copy(rows_v, out_hbm.at[pl.ds(base, bw)])

    return k


@jax.jit
def kernel(points, descriptors, mem_points, mem_descriptors, usage):
    del mem_descriptors
    b = points.shape[0]
    n = mem_points.shape[0]
    f = descriptors.shape[1]
    gd = (n + _NBD - 1) // _NBD
    npad = gd * _NBD
    gc = npad // _NBC

    psq = jnp.sum(points * points, axis=1)
    khalf = jnp.ceil(jnp.log2(jnp.max(psq) + 2.0) * 0.5)
    cshift = jnp.exp2(2.0 * khalf)
    mt = jnp.full((8, npad), 0.0, jnp.float32)
    mt = mt.at[:3, :n].set(mem_points.T).at[:3, n:].set(1e18)
    mt = mt.at[3, :].set(jnp.exp2(khalf))
    p8 = jnp.zeros((b, 8), jnp.float32).at[:, :3].set(points * -2.0)
    u_r = jnp.full((npad,), _IMAX, jnp.int32).at[:n].set(usage).reshape(
        gc, 1, _NBC)
    iota_c = jnp.arange(_NBD, dtype=jnp.int32)[None, :]
    cin = jnp.full((1, 128), cshift, jnp.float32)

    lw = pl.pallas_call(
        _fused_body,
        grid=(gd + 1,),
        in_specs=[
            pl.BlockSpec((b, 8), lambda i: (0, 0)),
            pl.BlockSpec((8, _NBD), lambda i: (0, jnp.minimum(i, gd - 1))),
            pl.BlockSpec((1, _NBD), lambda i: (0, 0)),
            pl.BlockSpec((gc, 1, _NBC), lambda i: (0, 0, 0)),
            pl.BlockSpec((1, 128), lambda i: (0, 0)),
        ],
        out_specs=pl.BlockSpec((b, 1), lambda i: (0, 0)),
        out_shape=jax.ShapeDtypeStruct((b, 1), jnp.int32),
        scratch_shapes=[
            pltpu.VMEM((b, 1), jnp.float32),
            pltpu.VMEM((b, 1), jnp.int32),
            pltpu.VMEM((b, 8), jnp.float32),
        ],
    )(p8, mt, iota_c, u_r, cin)

    return _sc_gather(b, f)(lw.reshape(b), descriptors)
